# baseline (speedup 1.0000x reference)
import functools

import jax
import jax.numpy as jnp
import numpy as np
from jax import lax
from jax.experimental import pallas as pl
from jax.experimental.pallas import tpu as pltpu
from jax.experimental.pallas import tpu_sc as plsc

_NDIM = 2
_KS = 3
_DIL = 0.05
_RADIUS = 0.1
_CIN = 32
_COUT = 32
_B = 2
_N = 2048
_NCELLS = _KS * _KS
_NSUB = 16
_QTC = 1408
_QSC = _N - _QTC
_QPT = _QSC // _NSUB
_NCH = _N // 16
_IT = 128
_INVH2 = 1.0 / (_RADIUS * _RADIUS)
_RU2 = (_RADIUS + _DIL * np.sqrt(2.0) + 1e-4) ** 2
_SIGMA = 40.0 / (7.0 * np.pi * _RADIUS * _RADIUS)

_P1 = [0.9991371839663212, -5.502522377118712, 22.393554226215347,
       -132.72180770441003, 657.0235949296153, -1802.1994576789311,
       2024.7410286026136]
_P2 = [1.0457526433688322, -5.765226799579174, 14.873446961611542,
       -22.908602965365144, 21.53951508925463, -11.30223631051044,
       2.517458213643078]
_P1S = [c * _SIGMA for c in _P1]
_P2S = [c * _SIGMA for c in _P2]


def _offset_list():
    c = (_KS - 1) / 2.0
    offs = []
    for a in range(_KS):
        for b in range(_KS):
            offs.append(((a - c) * _DIL, (b - c) * _DIL))
    return offs


def _horner(coefs, t):
    r = jnp.full((16,), coefs[-1], dtype=jnp.float32)
    for c in coefs[-2::-1]:
        r = r * t + c
    return r


_WLC = _N + 16


def _sc_body(xs_hbm, ys_hbm, im_hbm, den_hbm, data_hbm, f_hbm,
             sx, sy, sim, sden, ss, sdata, js, wlw, wlj, fbuf, sem):
    b = lax.axis_index("c")
    t = lax.axis_index("s")
    base_p = b * _N
    qbase = t * _QPT

    pltpu.sync_copy(xs_hbm.at[pl.ds(base_p, _N)], sx)
    pltpu.sync_copy(ys_hbm.at[pl.ds(base_p, _N)], sy)
    pltpu.sync_copy(im_hbm.at[pl.ds(base_p, _N)], sim)
    pltpu.sync_copy(den_hbm.at[pl.ds(base_p, _N)], sden)
    pltpu.sync_copy(data_hbm.at[pl.ds(base_p * _CIN, _N * _CIN)], sdata)

    def scale_body(k, carry):
        u = sim[pl.ds(k * 16, 16)]
        v = sden[pl.ds(k * 16, 16)]
        ss[pl.ds(k * 16, 16)] = 1.0 / (u * v)
        return carry
    lax.fori_loop(0, _NCH, scale_body, 0)

    offs = _offset_list()
    ih = _INVH2
    cu = 2.0 * _DIL * ih

    def query_body(i, carry):
        iq = _QTC + qbase + i
        iq_splat = jnp.full((16,), 0, jnp.int32) + iq
        xq = plsc.load_gather(sx, [iq_splat])
        yq = plsc.load_gather(sy, [iq_splat])

        def prune(k, cnt):
            b16 = k * 16
            xv = sx[pl.ds(b16, 16)]
            yv = sy[pl.ds(b16, 16)]
            dx = xq - xv
            dy = yq - yv
            r2 = dx * dx + dy * dy
            m = r2 < _RU2
            jv = lax.iota(jnp.int32, 16) + b16
            plsc.store_compressed(js.at[pl.ds(cnt, 16)], jv, mask=m)
            return cnt + plsc.all_reduce_population_count(m)[0]

        cnt = lax.fori_loop(0, _NCH, prune, 0)
        nch = (cnt + 15) >> 4

        def abody(k, cs):
            cb = k * 16
            lanes = lax.iota(jnp.int32, 16) + cb
            lm = lanes < cnt
            jv = js[pl.ds(cb, 16)]
            jv = jnp.where(lm, jv, 0)
            xv = plsc.load_gather(sx, [jv])
            yv = plsc.load_gather(sy, [jv])
            sv = plsc.load_gather(ss, [jv])
            sv = jnp.where(lm, sv, 0.0)
            dx = xq - xv
            dy = yq - yv
            r2t = (dx * dx + dy * dy + 1e-12) * ih
            uu = dx * cu
            vv = dy * cu
            jb_vec = jv * _CIN
            ncs = []
            for c, (ox, oy) in enumerate(offs):
                tt = r2t + ((ox * ox + oy * oy) * ih)
                if ox > 0:
                    tt = tt + uu
                elif ox < 0:
                    tt = tt - uu
                if oy > 0:
                    tt = tt + vv
                elif oy < 0:
                    tt = tt - vv
                inside = tt < 1.0
                w = jnp.where(tt < 0.25, _horner(_P1S, tt), _horner(_P2S, tt))
                w = w * sv
                m = inside & lm
                plsc.store_compressed(
                    wlw.at[pl.ds(c * _WLC + cs[c], 16)], w, mask=m)
                plsc.store_compressed(
                    wlj.at[pl.ds(c * _WLC + cs[c], 16)], jb_vec, mask=m)
                ncs.append(cs[c] + plsc.all_reduce_population_count(m)[0])
            return tuple(ncs)

        counts = lax.fori_loop(0, nch, abody, (0,) * _NCELLS)

        half = (i & 1) * (_NCELLS * _CIN)

        @pl.when(i >= 2)
        def _wait_prev():
            pltpu.make_async_copy(
                f_hbm.at[pl.ds(0, _NCELLS * _CIN)],
                fbuf.at[pl.ds(half, _NCELLS * _CIN)], sem).wait()

        for c in range(_NCELLS):
            cntc = counts[c]
            nbc = (cntc + 15) >> 4

            def bbody(k, acc, _c=c, _cntc=cntc):
                cb = k * 16
                lanes = lax.iota(jnp.int32, 16) + cb
                lm2 = lanes < _cntc
                wch = wlw[pl.ds(_c * _WLC + cb, 16)]
                jch = wlj[pl.ds(_c * _WLC + cb, 16)]
                wch = jnp.where(lm2, wch, 0.0)
                jch = jnp.where(lm2, jch, 0)
                a0, a1 = acc
                for l in range(16):
                    jb = jch[l]
                    ws = wch[l]
                    a0 = a0 + ws * sdata[pl.ds(jb, 16)]
                    a1 = a1 + ws * sdata[pl.ds(jb + 16, 16)]
                return (a0, a1)

            zero = jnp.zeros((16,), jnp.float32)
            a0, a1 = lax.fori_loop(0, nbc, bbody, (zero, zero))
            fbuf[pl.ds(half + c * _CIN, 16)] = a0
            fbuf[pl.ds(half + c * _CIN + 16, 16)] = a1

        rowoff = (b * _QSC + qbase + i) * (_NCELLS * _CIN)
        pltpu.make_async_copy(
            fbuf.at[pl.ds(half, _NCELLS * _CIN)],
            f_hbm.at[pl.ds(rowoff, _NCELLS * _CIN)], sem).start()
        return carry

    lax.fori_loop(0, _QPT, query_body, 0)
    for _ in range(2):
        pltpu.make_async_copy(
            f_hbm.at[pl.ds(0, _NCELLS * _CIN)],
            fbuf.at[pl.ds(0, _NCELLS * _CIN)], sem).wait()


def _sc_stage(xs, ys, im, den, data2d):
    mesh = plsc.VectorSubcoreMesh(core_axis_name="c", subcore_axis_name="s")
    fn = functools.partial(
        pl.kernel,
        out_type=jax.ShapeDtypeStruct((_B * _QSC * _NCELLS * _CIN,), jnp.float32),
        mesh=mesh,
        compiler_params=pltpu.CompilerParams(needs_layout_passes=False),
        scratch_types=[
            pltpu.VMEM((_N,), jnp.float32),
            pltpu.VMEM((_N,), jnp.float32),
            pltpu.VMEM((_N,), jnp.float32),
            pltpu.VMEM((_N,), jnp.float32),
            pltpu.VMEM((_N,), jnp.float32),
            pltpu.VMEM((_N * _CIN,), jnp.float32),
            pltpu.VMEM((_N + 32,), jnp.int32),
            pltpu.VMEM((_NCELLS * _WLC,), jnp.float32),
            pltpu.VMEM((_NCELLS * _WLC,), jnp.int32),
            pltpu.VMEM((2 * _NCELLS * _CIN,), jnp.float32),
            pltpu.SemaphoreType.DMA,
        ],
    )(_sc_body)
    return fn(xs, ys, im, den, data2d)


def _tc_body(locs_ref, data_ref, density_ref, weight_ref, bias_ref, out_ref):
    it = pl.program_id(1)
    xs = locs_ref[0, :, 0]
    ys = locs_ref[0, :, 1]
    invm = locs_ref[0, :, 2]
    den = density_ref[0, 0, :]
    wd = data_ref[0] * (1.0 / (invm * den))[:, None]
    wd16 = wd.astype(jnp.bfloat16)

    qx = locs_ref[0, pl.ds(it * _IT, _IT), 0]
    qy = locs_ref[0, pl.ds(it * _IT, _IT), 1]
    dx = qx[:, None] - xs[None, :]
    dy = qy[:, None] - ys[None, :]
    r2 = dx * dx + dy * dy + 1e-12
    u = dx * (2.0 * _DIL)
    v = dy * (2.0 * _DIL)

    acc = jnp.zeros((_IT, _COUT), dtype=jnp.float32)
    for c, (ox, oy) in enumerate(_offset_list()):
        d2 = r2 + (ox * ox + oy * oy)
        if ox > 0:
            d2 = d2 + u
        elif ox < 0:
            d2 = d2 - u
        if oy > 0:
            d2 = d2 + v
        elif oy < 0:
            d2 = d2 - v
        q = jnp.sqrt(d2) * (1.0 / _RADIUS)
        a = jnp.maximum(1.0 - q, 0.0)
        bb = jnp.maximum(0.5 - q, 0.0)
        a3 = a * a * a
        b3 = bb * bb * bb
        w = (2.0 * _SIGMA) * (a3 - 4.0 * b3)
        f = jnp.dot(w.astype(jnp.bfloat16), wd16,
                    preferred_element_type=jnp.float32)
        acc = acc + jnp.dot(f, weight_ref[:, :, c].T,
                            preferred_element_type=jnp.float32)
    out_ref[0] = acc + bias_ref[:][None, :]


def _tc_dense(locs, data, density, weight, bias):
    grid = (_B, _QTC // _IT)
    return pl.pallas_call(
        _tc_body,
        grid=grid,
        in_specs=[
            pl.BlockSpec((1, _N, _NDIM + 1), lambda b, i: (b, 0, 0)),
            pl.BlockSpec((1, _N, _CIN), lambda b, i: (b, 0, 0)),
            pl.BlockSpec((1, 1, _N), lambda b, i: (b, 0, 0)),
            pl.BlockSpec((_COUT, _CIN, _NCELLS), lambda b, i: (0, 0, 0)),
            pl.BlockSpec((_COUT,), lambda b, i: (0,)),
        ],
        out_specs=pl.BlockSpec((1, _IT, _COUT), lambda b, i: (b, i, 0)),
        out_shape=jax.ShapeDtypeStruct((_B, _QTC, _COUT), jnp.float32),
    )(locs, data, density.reshape(_B, 1, _N), weight, bias)


def _proj_body(f_ref, w_ref, b_ref, o_ref):
    o_ref[...] = jnp.dot(f_ref[...], w_ref[...],
                         preferred_element_type=jnp.float32) + b_ref[0][None, :]


def _project(f2d, w2d, bias):
    return pl.pallas_call(
        _proj_body,
        out_shape=jax.ShapeDtypeStruct((_B * _QSC, _COUT), jnp.float32),
    )(f2d, w2d, bias.reshape(1, _COUT))


@jax.jit
def kernel(locs, data, density, weight, bias):
    xs = locs[..., 0].reshape(_B * _N)
    ys = locs[..., 1].reshape(_B * _N)
    im = locs[..., 2].reshape(_B * _N)
    den = density.reshape(_B * _N)
    data2d = data.reshape(_B * _N * _CIN)
    f_flat = _sc_stage(xs, ys, im, den, data2d)
    out_tc = _tc_dense(locs, data, density, weight, bias)
    f2d = f_flat.reshape(_B * _QSC, _NCELLS * _CIN)
    w2d = jnp.transpose(weight, (2, 1, 0)).reshape(_NCELLS * _CIN, _COUT)
    out_sc = _project(f2d, w2d, bias).reshape(_B, _QSC, _COUT)
    return jnp.concatenate([out_tc, out_sc], axis=1)


# speedup vs baseline: 1.1592x; 1.1592x over previous
import functools

import jax
import jax.numpy as jnp
import numpy as np
from jax import lax
from jax.experimental import pallas as pl
from jax.experimental.pallas import tpu as pltpu
from jax.experimental.pallas import tpu_sc as plsc

_NDIM = 2
_KS = 3
_DIL = 0.05
_RADIUS = 0.1
_CIN = 32
_COUT = 32
_B = 2
_N = 2048
_NCELLS = _KS * _KS
_NSUB = 16
_QTC = 1280
_QSC = _N - _QTC
_QPT = _QSC // _NSUB
_NCH = _N // 16
_IT = 256
_INVH2 = 1.0 / (_RADIUS * _RADIUS)
_RU2 = (_RADIUS + _DIL * np.sqrt(2.0) + 1e-4) ** 2
_SIGMA = 40.0 / (7.0 * np.pi * _RADIUS * _RADIUS)

_P1 = [0.9991371839663212, -5.502522377118712, 22.393554226215347,
       -132.72180770441003, 657.0235949296153, -1802.1994576789311,
       2024.7410286026136]
_P2 = [1.0457526433688322, -5.765226799579174, 14.873446961611542,
       -22.908602965365144, 21.53951508925463, -11.30223631051044,
       2.517458213643078]
_P1S = [c * _SIGMA for c in _P1]
_P2S = [c * _SIGMA for c in _P2]


def _offset_list():
    c = (_KS - 1) / 2.0
    offs = []
    for a in range(_KS):
        for b in range(_KS):
            offs.append(((a - c) * _DIL, (b - c) * _DIL))
    return offs


def _horner(coefs, t):
    r = jnp.full((16,), coefs[-1], dtype=jnp.float32)
    for c in coefs[-2::-1]:
        r = r * t + c
    return r


_WLC = _N + 16


def _sc_body(xs_hbm, ys_hbm, im_hbm, den_hbm, data_hbm, f_hbm,
             sx, sy, sim, sden, ss, sdata, js, wlw, wlj, fbuf, sem):
    b = lax.axis_index("c")
    t = lax.axis_index("s")
    base_p = b * _N
    qbase = t * _QPT

    pltpu.sync_copy(xs_hbm.at[pl.ds(base_p, _N)], sx)
    pltpu.sync_copy(ys_hbm.at[pl.ds(base_p, _N)], sy)
    pltpu.sync_copy(im_hbm.at[pl.ds(base_p, _N)], sim)
    pltpu.sync_copy(den_hbm.at[pl.ds(base_p, _N)], sden)
    pltpu.sync_copy(data_hbm.at[pl.ds(base_p * _CIN, _N * _CIN)], sdata)

    def scale_body(k, carry):
        u = sim[pl.ds(k * 16, 16)]
        v = sden[pl.ds(k * 16, 16)]
        ss[pl.ds(k * 16, 16)] = 1.0 / (u * v)
        return carry
    lax.fori_loop(0, _NCH, scale_body, 0)

    offs = _offset_list()
    ih = _INVH2
    cu = 2.0 * _DIL * ih

    def query_body(i, carry):
        iq = _QTC + qbase + i
        iq_splat = jnp.full((16,), 0, jnp.int32) + iq
        xq = plsc.load_gather(sx, [iq_splat])
        yq = plsc.load_gather(sy, [iq_splat])

        def prune(k, cnt):
            b16 = k * 16
            xv = sx[pl.ds(b16, 16)]
            yv = sy[pl.ds(b16, 16)]
            dx = xq - xv
            dy = yq - yv
            r2 = dx * dx + dy * dy
            m = r2 < _RU2
            jv = lax.iota(jnp.int32, 16) + b16
            plsc.store_compressed(js.at[pl.ds(cnt, 16)], jv, mask=m)
            return cnt + plsc.all_reduce_population_count(m)[0]

        cnt = lax.fori_loop(0, _NCH, prune, 0)
        nch = (cnt + 15) >> 4

        def abody(k, cs):
            cb = k * 16
            lanes = lax.iota(jnp.int32, 16) + cb
            lm = lanes < cnt
            jv = js[pl.ds(cb, 16)]
            jv = jnp.where(lm, jv, 0)
            xv = plsc.load_gather(sx, [jv])
            yv = plsc.load_gather(sy, [jv])
            sv = plsc.load_gather(ss, [jv])
            sv = jnp.where(lm, sv, 0.0)
            dx = xq - xv
            dy = yq - yv
            r2t = (dx * dx + dy * dy + 1e-12) * ih
            uu = dx * cu
            vv = dy * cu
            jb_vec = jv * _CIN
            ncs = []
            for c, (ox, oy) in enumerate(offs):
                tt = r2t + ((ox * ox + oy * oy) * ih)
                if ox > 0:
                    tt = tt + uu
                elif ox < 0:
                    tt = tt - uu
                if oy > 0:
                    tt = tt + vv
                elif oy < 0:
                    tt = tt - vv
                inside = tt < 1.0
                w = jnp.where(tt < 0.25, _horner(_P1S, tt), _horner(_P2S, tt))
                w = w * sv
                m = inside & lm
                plsc.store_compressed(
                    wlw.at[pl.ds(c * _WLC + cs[c], 16)], w, mask=m)
                plsc.store_compressed(
                    wlj.at[pl.ds(c * _WLC + cs[c], 16)], jb_vec, mask=m)
                ncs.append(cs[c] + plsc.all_reduce_population_count(m)[0])
            return tuple(ncs)

        counts = lax.fori_loop(0, nch, abody, (0,) * _NCELLS)

        half = (i & 1) * (_NCELLS * _CIN)

        @pl.when(i >= 2)
        def _wait_prev():
            pltpu.make_async_copy(
                f_hbm.at[pl.ds(0, _NCELLS * _CIN)],
                fbuf.at[pl.ds(half, _NCELLS * _CIN)], sem).wait()

        for c in range(_NCELLS):
            cntc = counts[c]
            nbc = (cntc + 15) >> 4

            def bbody(k, acc, _c=c, _cntc=cntc):
                cb = k * 16
                lanes = lax.iota(jnp.int32, 16) + cb
                lm2 = lanes < _cntc
                wch = wlw[pl.ds(_c * _WLC + cb, 16)]
                jch = wlj[pl.ds(_c * _WLC + cb, 16)]
                wch = jnp.where(lm2, wch, 0.0)
                jch = jnp.where(lm2, jch, 0)
                a0, a1 = acc
                for l in range(16):
                    jb = jch[l]
                    ws = wch[l]
                    a0 = a0 + ws * sdata[pl.ds(jb, 16)]
                    a1 = a1 + ws * sdata[pl.ds(jb + 16, 16)]
                return (a0, a1)

            zero = jnp.zeros((16,), jnp.float32)
            a0, a1 = lax.fori_loop(0, nbc, bbody, (zero, zero))
            fbuf[pl.ds(half + c * _CIN, 16)] = a0
            fbuf[pl.ds(half + c * _CIN + 16, 16)] = a1

        rowoff = (b * _QSC + qbase + i) * (_NCELLS * _CIN)
        pltpu.make_async_copy(
            fbuf.at[pl.ds(half, _NCELLS * _CIN)],
            f_hbm.at[pl.ds(rowoff, _NCELLS * _CIN)], sem).start()
        return carry

    lax.fori_loop(0, _QPT, query_body, 0)
    for _ in range(2):
        pltpu.make_async_copy(
            f_hbm.at[pl.ds(0, _NCELLS * _CIN)],
            fbuf.at[pl.ds(0, _NCELLS * _CIN)], sem).wait()


def _sc_stage(xs, ys, im, den, data2d):
    mesh = plsc.VectorSubcoreMesh(core_axis_name="c", subcore_axis_name="s")
    fn = functools.partial(
        pl.kernel,
        out_type=jax.ShapeDtypeStruct((_B * _QSC * _NCELLS * _CIN,), jnp.float32),
        mesh=mesh,
        compiler_params=pltpu.CompilerParams(needs_layout_passes=False),
        scratch_types=[
            pltpu.VMEM((_N,), jnp.float32),
            pltpu.VMEM((_N,), jnp.float32),
            pltpu.VMEM((_N,), jnp.float32),
            pltpu.VMEM((_N,), jnp.float32),
            pltpu.VMEM((_N,), jnp.float32),
            pltpu.VMEM((_N * _CIN,), jnp.float32),
            pltpu.VMEM((_N + 32,), jnp.int32),
            pltpu.VMEM((_NCELLS * _WLC,), jnp.float32),
            pltpu.VMEM((_NCELLS * _WLC,), jnp.int32),
            pltpu.VMEM((2 * _NCELLS * _CIN,), jnp.float32),
            pltpu.SemaphoreType.DMA,
        ],
    )(_sc_body)
    return fn(xs, ys, im, den, data2d)


def _tc_body(locs_ref, data_ref, density_ref, weight_ref, bias_ref, out_ref):
    it = pl.program_id(1)
    xs = locs_ref[0, :, 0]
    ys = locs_ref[0, :, 1]
    invm = locs_ref[0, :, 2]
    den = density_ref[0, 0, :]
    wd = data_ref[0] * (1.0 / (invm * den))[:, None]
    wd16 = wd.astype(jnp.bfloat16)

    qx = locs_ref[0, pl.ds(it * _IT, _IT), 0]
    qy = locs_ref[0, pl.ds(it * _IT, _IT), 1]
    dx = qx[:, None] - xs[None, :]
    dy = qy[:, None] - ys[None, :]
    r2 = dx * dx + dy * dy + 1e-12
    u = dx * (2.0 * _DIL)
    v = dy * (2.0 * _DIL)

    acc = jnp.zeros((_IT, _COUT), dtype=jnp.float32)
    for c, (ox, oy) in enumerate(_offset_list()):
        d2 = r2 + (ox * ox + oy * oy)
        if ox > 0:
            d2 = d2 + u
        elif ox < 0:
            d2 = d2 - u
        if oy > 0:
            d2 = d2 + v
        elif oy < 0:
            d2 = d2 - v
        q = jnp.sqrt(d2) * (1.0 / _RADIUS)
        a = jnp.maximum(1.0 - q, 0.0)
        bb = jnp.maximum(0.5 - q, 0.0)
        a3 = a * a * a
        b3 = bb * bb * bb
        w = (2.0 * _SIGMA) * (a3 - 4.0 * b3)
        f = jnp.dot(w.astype(jnp.bfloat16), wd16,
                    preferred_element_type=jnp.float32)
        acc = acc + jnp.dot(f, weight_ref[:, :, c].T,
                            preferred_element_type=jnp.float32)
    out_ref[0] = acc + bias_ref[:][None, :]


def _tc_dense(locs, data, density, weight, bias):
    grid = (_B, _QTC // _IT)
    return pl.pallas_call(
        _tc_body,
        grid=grid,
        in_specs=[
            pl.BlockSpec((1, _N, _NDIM + 1), lambda b, i: (b, 0, 0)),
            pl.BlockSpec((1, _N, _CIN), lambda b, i: (b, 0, 0)),
            pl.BlockSpec((1, 1, _N), lambda b, i: (b, 0, 0)),
            pl.BlockSpec((_COUT, _CIN, _NCELLS), lambda b, i: (0, 0, 0)),
            pl.BlockSpec((_COUT,), lambda b, i: (0,)),
        ],
        out_specs=pl.BlockSpec((1, _IT, _COUT), lambda b, i: (b, i, 0)),
        out_shape=jax.ShapeDtypeStruct((_B, _QTC, _COUT), jnp.float32),
    )(locs, data, density.reshape(_B, 1, _N), weight, bias)


def _proj_body(f_ref, w_ref, b_ref, o_ref):
    o_ref[...] = jnp.dot(f_ref[...], w_ref[...],
                         preferred_element_type=jnp.float32) + b_ref[0][None, :]


def _project(f2d, w2d, bias):
    return pl.pallas_call(
        _proj_body,
        out_shape=jax.ShapeDtypeStruct((_B * _QSC, _COUT), jnp.float32),
    )(f2d, w2d, bias.reshape(1, _COUT))


@jax.jit
def kernel(locs, data, density, weight, bias):
    xs = locs[..., 0].reshape(_B * _N)
    ys = locs[..., 1].reshape(_B * _N)
    im = locs[..., 2].reshape(_B * _N)
    den = density.reshape(_B * _N)
    data2d = data.reshape(_B * _N * _CIN)
    f_flat = _sc_stage(xs, ys, im, den, data2d)
    out_tc = _tc_dense(locs, data, density, weight, bias)
    f2d = f_flat.reshape(_B * _QSC, _NCELLS * _CIN)
    w2d = jnp.transpose(weight, (2, 1, 0)).reshape(_NCELLS * _CIN, _COUT)
    out_sc = _project(f2d, w2d, bias).reshape(_B, _QSC, _COUT)
    return jnp.concatenate([out_tc, out_sc], axis=1)
